# Initial kernel scaffold; baseline (speedup 1.0000x reference)
import jax
import jax.numpy as jnp
from jax.experimental import pallas as pl


def kernel(mem, x, n_id, batch_size):
    raise NotImplementedError("write your pallas kernel here")


# speedup vs baseline: 2.8175x; 2.8175x over previous
import jax
import jax.numpy as jnp
from jax import lax
from jax.experimental import pallas as pl
from jax.experimental.pallas import tpu as pltpu
from jax.experimental.pallas import tpu_sc as plsc

_BS = 65536
_TOTAL = 131072
_H = 128
_TBL = 1 << 20
_NW = 32
_HPW = (_TOTAL - _BS) // _NW
_CH = 64
_NCH = _HPW // _CH
_NP = 2
_SLICE = _TBL // (16 * _NP)
_SH = 15
_IDC = 2048


def _body(mem_hbm, x_hbm, nid_hbm, delta_hbm, out_hbm,
          table_v, ids_v, dbuf_v, gbuf_v, kbuf_v, kcl_v, rows_v, xrows_v,
          spmem_tbl):
    c = lax.axis_index("c")
    s = lax.axis_index("s")
    wid = s * 2 + c

    iota16 = lax.iota(jnp.int32, 16)
    neg1 = jnp.full((16,), -1, jnp.int32)

    pltpu.sync_copy(delta_hbm, dbuf_v)
    dv = dbuf_v[...]

    for p in range(_NP):
        rng = _NP * s + p

        @pl.loop(0, _SLICE // 16)
        def _init(i):
            table_v[pl.ds(i * 16, 16)] = neg1

        @pl.loop(0, _BS // _IDC)
        def _scan(chunk):
            pltpu.sync_copy(nid_hbm.at[pl.ds(chunk * _IDC, _IDC)], ids_v)

            @pl.loop(0, _IDC // 16)
            def _vec(i):
                g = ids_v[pl.ds(i * 16, 16)] + dv
                mask = lax.shift_right_arithmetic(g, _SH) == rng
                loc = lax.bitwise_and(g, jnp.int32(_SLICE - 1))
                kv = iota16 + (chunk * _IDC + i * 16)
                plsc.store_scatter(table_v, [loc], kv, mask=mask)

        pltpu.sync_copy(table_v, spmem_tbl.at[pl.ds(rng * _SLICE, _SLICE)])

    @pl.loop(0, _BS // _NW // _CH)
    def _xcopy(i):
        base = wid * (_BS // _NW) + i * _CH
        pltpu.sync_copy(x_hbm.at[pl.ds(base, _CH)], rows_v)
        pltpu.sync_copy(rows_v, out_hbm.at[pl.ds(base, _CH)])

    plsc.subcore_barrier()

    @pl.loop(0, _NCH)
    def _halo_idx(cc):
        pltpu.sync_copy(
            nid_hbm.at[pl.ds(_BS + wid * _HPW + cc * _CH, _CH)], gbuf_v.at[cc])
        pltpu.sync_copy(spmem_tbl.at[gbuf_v.at[cc]], kbuf_v.at[cc])

    @pl.loop(0, _NCH)
    def _halo(cc):
        pltpu.sync_copy(mem_hbm.at[gbuf_v.at[cc]], rows_v)

        @pl.loop(0, _CH // 16)
        def _clamp(i):
            kcl_v[pl.ds(i * 16, 16)] = jnp.maximum(
                kbuf_v[cc, pl.ds(i * 16, 16)], 0)

        pltpu.sync_copy(x_hbm.at[kcl_v], xrows_v)

        @pl.loop(0, _CH // 16)
        def _fix(rg):
            kv = kbuf_v[cc, pl.ds(rg * 16, 16)]
            for j in range(16):
                k = kv[j]
                r = rg * 16 + j

                @pl.when(k >= 0)
                def _():
                    @pl.loop(0, _H // 16)
                    def _cp(v):
                        rows_v[r, pl.ds(v * 16, 16)] = (
                            xrows_v[r, pl.ds(v * 16, 16)])

        pltpu.sync_copy(
            rows_v, out_hbm.at[pl.ds(_BS + wid * _HPW + cc * _CH, _CH)])


def kernel(mem, x, n_id, batch_size):
    delta = jnp.full((16,), batch_size - _BS, dtype=jnp.int32)
    mesh = plsc.VectorSubcoreMesh(core_axis_name="c", subcore_axis_name="s")
    return pl.kernel(
        _body,
        out_type=jax.ShapeDtypeStruct((_TOTAL, _H), jnp.float32),
        mesh=mesh,
        compiler_params=pltpu.CompilerParams(needs_layout_passes=False),
        scratch_types=[
            pltpu.VMEM((_SLICE,), jnp.int32),
            pltpu.VMEM((_IDC,), jnp.int32),
            pltpu.VMEM((16,), jnp.int32),
            pltpu.VMEM((_NCH, _CH), jnp.int32),
            pltpu.VMEM((_NCH, _CH), jnp.int32),
            pltpu.VMEM((_CH,), jnp.int32),
            pltpu.VMEM((_CH, _H), jnp.float32),
            pltpu.VMEM((_CH, _H), jnp.float32),
            pltpu.VMEM_SHARED((_TBL,), jnp.int32),
        ],
    )(mem, x, n_id, delta)


# speedup vs baseline: 6.3798x; 2.2643x over previous
import jax
import jax.numpy as jnp
from jax import lax
from jax.experimental import pallas as pl
from jax.experimental.pallas import tpu as pltpu
from jax.experimental.pallas import tpu_sc as plsc

_BS = 65536
_TOTAL = 131072
_H = 128
_TBL = 1 << 20
_NW = 32
_HPW = (_TOTAL - _BS) // _NW
_CH = 64
_NCH = _HPW // _CH
_NP = 2
_SLICE = _TBL // (16 * _NP)
_SH = 15
_IDC = 2048
_FXC = _HPW + _CH


def _body(mem_hbm, x_hbm, nid_hbm, delta_hbm, out_hbm,
          table_v, ids_v, dbuf_v, gbuf_v, kbuf_v, jc_v, kc_v, jidx_v, rows_v,
          spmem_tbl, semg, semw, semx):
    c = lax.axis_index("c")
    s = lax.axis_index("s")
    wid = s * 2 + c

    iota16 = lax.iota(jnp.int32, 16)
    neg1 = jnp.full((16,), -1, jnp.int32)

    pltpu.sync_copy(delta_hbm, dbuf_v)
    dv = dbuf_v[...]

    for p in range(_NP):
        rng = _NP * s + p

        @pl.loop(0, _SLICE // 16)
        def _init(i):
            table_v[pl.ds(i * 16, 16)] = neg1

        @pl.loop(0, _BS // _IDC)
        def _scan(chunk):
            pltpu.sync_copy(nid_hbm.at[pl.ds(chunk * _IDC, _IDC)], ids_v)

            @pl.loop(0, _IDC // 16)
            def _vec(i):
                g = ids_v[pl.ds(i * 16, 16)] + dv
                mask = lax.shift_right_arithmetic(g, _SH) == rng
                loc = lax.bitwise_and(g, jnp.int32(_SLICE - 1))
                kv = iota16 + (chunk * _IDC + i * 16)
                plsc.store_scatter(table_v, [loc], kv, mask=mask)

        pltpu.sync_copy(table_v, spmem_tbl.at[pl.ds(rng * _SLICE, _SLICE)])

    xbase = wid * (_BS // _NW)
    pltpu.async_copy(x_hbm.at[pl.ds(xbase, _BS // _NW)],
                     out_hbm.at[pl.ds(xbase, _BS // _NW)], semx)

    plsc.subcore_barrier()

    hbase = _BS + wid * _HPW
    pltpu.sync_copy(nid_hbm.at[pl.ds(hbase, _HPW)], gbuf_v)

    @pl.loop(0, _NCH)
    def _kgather(cc):
        pltpu.sync_copy(spmem_tbl.at[gbuf_v.at[pl.ds(cc * _CH, _CH)]],
                        kbuf_v.at[pl.ds(cc * _CH, _CH)])

    def _g(cc):
        return (mem_hbm.at[gbuf_v.at[pl.ds(cc * _CH, _CH)]],
                rows_v.at[cc % 2])

    def _w(cc):
        return (rows_v.at[cc % 2], out_hbm.at[pl.ds(hbase + cc * _CH, _CH)])

    @pl.loop(0, _HPW // 16, init_carry=jnp.int32(0))
    def _compact(i, off):
        kv = kbuf_v[pl.ds(i * 16, 16)]
        mask = kv >= 0
        jv = iota16 + (hbase + i * 16)
        plsc.store_compressed(kc_v.at[pl.ds(off, 16)], kv, mask=mask)
        plsc.store_compressed(jc_v.at[pl.ds(off, 16)], jv, mask=mask)
        return off + plsc.all_reduce_population_count(mask)[0]

    nh = _compact

    pltpu.async_copy(*_g(0), semg)
    for cc in range(_NCH):
        if cc + 1 < _NCH:
            if cc - 1 >= 0:
                pltpu.make_async_copy(*_w(cc - 1), semw).wait()
            pltpu.async_copy(*_g(cc + 1), semg)
        pltpu.make_async_copy(*_g(cc), semg).wait()
        pltpu.async_copy(*_w(cc), semw)
    pltpu.make_async_copy(*_w(_NCH - 2), semw).wait()
    pltpu.make_async_copy(*_w(_NCH - 1), semw).wait()

    pltpu.make_async_copy(x_hbm.at[pl.ds(xbase, _BS // _NW)],
                          out_hbm.at[pl.ds(xbase, _BS // _NW)], semx).wait()

    @pl.when(nh > 0)
    def _():
        k0 = jnp.broadcast_to(kc_v[pl.ds(0, 16)][0], (16,))
        j0 = jnp.broadcast_to(jc_v[pl.ds(0, 16)][0], (16,))
        pad_end = ((nh + _CH - 1) // _CH) * _CH

        @pl.loop(nh // 16 * 16, pad_end, step=16)
        def _pad(pos):
            idx = iota16 + pos
            keep = idx < nh
            kc_v[pl.ds(pos, 16)] = jnp.where(keep, kc_v[pl.ds(pos, 16)], k0)
            jc_v[pl.ds(pos, 16)] = jnp.where(keep, jc_v[pl.ds(pos, 16)], j0)

        @pl.loop(0, (nh + _CH - 1) // _CH)
        def _fix(cc):
            pltpu.sync_copy(x_hbm.at[kc_v.at[pl.ds(cc * _CH, _CH)]],
                            rows_v.at[0])
            for i in range(_CH // 16):
                jidx_v[0, pl.ds(i * 16, 16)] = (
                    jc_v[pl.ds(cc * _CH + i * 16, 16)])
            pltpu.sync_copy(rows_v.at[0], out_hbm.at[jidx_v.at[0]])


def kernel(mem, x, n_id, batch_size):
    delta = jnp.full((16,), batch_size - _BS, dtype=jnp.int32)
    mesh = plsc.VectorSubcoreMesh(core_axis_name="c", subcore_axis_name="s")
    return pl.kernel(
        _body,
        out_type=jax.ShapeDtypeStruct((_TOTAL, _H), jnp.float32),
        mesh=mesh,
        compiler_params=pltpu.CompilerParams(needs_layout_passes=False),
        scratch_types=[
            pltpu.VMEM((_SLICE,), jnp.int32),
            pltpu.VMEM((_IDC,), jnp.int32),
            pltpu.VMEM((16,), jnp.int32),
            pltpu.VMEM((_HPW,), jnp.int32),
            pltpu.VMEM((_HPW,), jnp.int32),
            pltpu.VMEM((_FXC,), jnp.int32),
            pltpu.VMEM((_FXC,), jnp.int32),
            pltpu.VMEM((1, _CH), jnp.int32),
            pltpu.VMEM((2, _CH, _H), jnp.float32),
            pltpu.VMEM_SHARED((_TBL,), jnp.int32),
            pltpu.SemaphoreType.DMA,
            pltpu.SemaphoreType.DMA,
            pltpu.SemaphoreType.DMA,
        ],
    )(mem, x, n_id, delta)
